# Initial kernel scaffold; baseline (speedup 1.0000x reference)
import jax
import jax.numpy as jnp
from jax.experimental import pallas as pl


def kernel(token_states, memory_tokens, src_ids):
    raise NotImplementedError("write your pallas kernel here")


# speedup vs baseline: 5.0251x; 5.0251x over previous
import functools

import jax
import jax.numpy as jnp
from jax.experimental import pallas as pl
from jax.experimental.pallas import tpu as pltpu

WINDOW = 128
STRIDE = 64
SEQ_BLOCK = 512


def _pooled_broadcast_kernel(mem_ref, repr_ref, w_ref, acc_ref, *,
                             num_seq_blocks, edge, inv_norm, inv_sets):
    s = pl.program_id(1)
    block = mem_ref[0]
    colsum = jnp.sum(block, axis=0, keepdims=True)
    partial = colsum + colsum

    @pl.when(s == 0)
    def _init():
        acc_ref[...] = partial - jnp.sum(block[:edge], axis=0, keepdims=True)

    @pl.when(s != 0)
    def _accum():
        extra = jnp.where(
            s == num_seq_blocks - 1,
            jnp.sum(block[SEQ_BLOCK - edge:], axis=0, keepdims=True),
            jnp.zeros_like(colsum),
        )
        acc_ref[...] = acc_ref[...] + partial - extra

    @pl.when(s == num_seq_blocks - 1)
    def _finalize():
        r = acc_ref[...] * inv_norm
        repr_ref[0] = jnp.broadcast_to(r, repr_ref.shape[1:])
        w_ref[...] = jnp.full(w_ref.shape, inv_sets, dtype=w_ref.dtype)


def kernel(token_states, memory_tokens, src_ids):
    batch, seq_len, d = memory_tokens.shape
    num_tokens = token_states.shape[1]
    num_sets = (seq_len - WINDOW) // STRIDE + 1
    edge = WINDOW - STRIDE
    num_seq_blocks = seq_len // SEQ_BLOCK

    body = functools.partial(
        _pooled_broadcast_kernel,
        num_seq_blocks=num_seq_blocks,
        edge=edge,
        inv_norm=1.0 / (num_sets * WINDOW),
        inv_sets=1.0 / num_sets,
    )

    token_repr, weights = pl.pallas_call(
        body,
        grid=(batch, num_seq_blocks),
        in_specs=[
            pl.BlockSpec((1, SEQ_BLOCK, d), lambda b, s: (b, s, 0)),
        ],
        out_specs=[
            pl.BlockSpec((1, num_tokens, d), lambda b, s: (b, 0, 0)),
            pl.BlockSpec((1, num_tokens, num_sets), lambda b, s: (b, 0, 0)),
        ],
        out_shape=[
            jax.ShapeDtypeStruct((batch, num_tokens, d), jnp.float32),
            jax.ShapeDtypeStruct((batch, num_tokens, num_sets), jnp.float32),
        ],
        scratch_shapes=[pltpu.VMEM((1, d), jnp.float32)],
    )(memory_tokens)

    return (token_repr, weights)
